# Initial kernel scaffold; baseline (speedup 1.0000x reference)
import jax
import jax.numpy as jnp
from jax.experimental import pallas as pl


def kernel(x, edge_index, in_W, in_b, l0_W, l0_b, nt0_W, nt0_b, a0, l1_W, l1_b, nt1_W, nt1_b, a1, fe0_W, fe0_b, out_W, out_b):
    raise NotImplementedError("write your pallas kernel here")


# speedup vs baseline: 2.5324x; 2.5324x over previous
import functools

import jax
import jax.numpy as jnp
from jax import lax
from jax.experimental import pallas as pl
from jax.experimental.pallas import tpu as pltpu
from jax.experimental.pallas import tpu_sc as plsc

N = 10000
E = 160000
D = 256
DH = 128
E_PAD = 163840
N_ACC = 10112
ZSPAN = 632
OSPAN = 632
OSPAN_LAST = N - 15 * OSPAN
CHUNK = 128
SEG_CHUNKS = E_PAD // 16 // CHUNK
CNT_CHUNKS = E_PAD // 32 // CHUNK
CNT_W = 128

BN = 400
GRID = N // BN

_mesh = plsc.VectorSubcoreMesh(core_axis_name="c", subcore_axis_name="s")


@functools.partial(
    pl.kernel,
    mesh=_mesh,
    out_type=[
        jax.ShapeDtypeStruct((N, DH), jnp.float32),
        jax.ShapeDtypeStruct((N, DH), jnp.float32),
    ],
    scratch_types=[
        pltpu.VMEM_SHARED((N_ACC, DH), jnp.float32),
        pltpu.VMEM((CHUNK,), jnp.int32),
        pltpu.VMEM((CHUNK,), jnp.int32),
        pltpu.VMEM((CHUNK, DH), jnp.float32),
        pltpu.SemaphoreType.DMA,
    ],
)
def _segsum(xt_lo, xt_hi, row_hbm, col_hbm, s_lo, s_hi,
            acc, row_buf, col_buf, gbuf, sem):
    c = lax.axis_index("c")
    sid = lax.axis_index("s")

    def _zrow(i, carry):
        for j in range(DH // 16):
            gbuf[i, pl.ds(j * 16, 16)] = jnp.zeros((16,), jnp.float32)
        return carry
    lax.fori_loop(0, CHUNK, _zrow, 0)
    zbase = sid * ZSPAN
    for k in range((ZSPAN + CHUNK - 1) // CHUNK):
        n = min(CHUNK, ZSPAN - k * CHUNK)
        pltpu.sync_copy(gbuf.at[pl.ds(0, n)],
                        acc.at[pl.ds(zbase + k * CHUNK, n)])
    plsc.subcore_barrier()

    def _edge_loop(table):
        ebase = sid * (E_PAD // 16)
        def _body(ch, carry):
            b = ebase + ch * CHUNK
            pltpu.sync_copy(row_hbm.at[pl.ds(b, CHUNK)], row_buf)
            pltpu.sync_copy(col_hbm.at[pl.ds(b, CHUNK)], col_buf)
            pltpu.async_copy(table.at[row_buf], gbuf, sem).wait()
            pltpu.sync_copy(gbuf, acc.at[col_buf], add=True)
            return carry
        lax.fori_loop(0, SEG_CHUNKS, _body, 0)

    @pl.when(c == 0)
    def _():
        _edge_loop(xt_lo)

    @pl.when(c == 1)
    def _():
        _edge_loop(xt_hi)

    plsc.subcore_barrier()
    _write_out(c, sid, acc, s_lo, s_hi)


def _write_out(c, sid, acc, out0, out1):
    obase = sid * OSPAN

    @pl.when((c == 0) & (sid < 15))
    def _():
        pltpu.sync_copy(acc.at[pl.ds(obase, OSPAN)],
                        out0.at[pl.ds(obase, OSPAN)])

    @pl.when((c == 0) & (sid == 15))
    def _():
        pltpu.sync_copy(acc.at[pl.ds(15 * OSPAN, OSPAN_LAST)],
                        out0.at[pl.ds(15 * OSPAN, OSPAN_LAST)])

    @pl.when((c == 1) & (sid < 15))
    def _():
        pltpu.sync_copy(acc.at[pl.ds(obase, OSPAN)],
                        out1.at[pl.ds(obase, OSPAN)])

    @pl.when((c == 1) & (sid == 15))
    def _():
        pltpu.sync_copy(acc.at[pl.ds(15 * OSPAN, OSPAN_LAST)],
                        out1.at[pl.ds(15 * OSPAN, OSPAN_LAST)])


@functools.partial(
    pl.kernel,
    mesh=_mesh,
    out_type=[
        jax.ShapeDtypeStruct((N, CNT_W), jnp.float32),
        jax.ShapeDtypeStruct((N, CNT_W), jnp.float32),
    ],
    scratch_types=[
        pltpu.VMEM_SHARED((N_ACC, CNT_W), jnp.float32),
        pltpu.VMEM((CHUNK,), jnp.int32),
        pltpu.VMEM((CHUNK, CNT_W), jnp.float32),
        pltpu.VMEM((CHUNK, CNT_W), jnp.float32),
    ],
)
def _cnt(col_hbm, cnt_a, cnt_b, acc, col_buf, ones_buf, zbuf):
    c = lax.axis_index("c")
    sid = lax.axis_index("s")

    def _fill(i, carry):
        for j in range(CNT_W // 16):
            ones_buf[i, pl.ds(j * 16, 16)] = jnp.ones((16,), jnp.float32)
            zbuf[i, pl.ds(j * 16, 16)] = jnp.zeros((16,), jnp.float32)
        return carry
    lax.fori_loop(0, CHUNK, _fill, 0)
    zbase = sid * ZSPAN
    for k in range((ZSPAN + CHUNK - 1) // CHUNK):
        n = min(CHUNK, ZSPAN - k * CHUNK)
        pltpu.sync_copy(zbuf.at[pl.ds(0, n)],
                        acc.at[pl.ds(zbase + k * CHUNK, n)])
    plsc.subcore_barrier()

    wid = sid * 2 + c
    ebase = wid * (E_PAD // 32)
    def _body(ch, carry):
        b = ebase + ch * CHUNK
        pltpu.sync_copy(col_hbm.at[pl.ds(b, CHUNK)], col_buf)
        pltpu.sync_copy(ones_buf, acc.at[col_buf], add=True)
        return carry
    lax.fori_loop(0, CNT_CHUNKS, _body, 0)

    plsc.subcore_barrier()
    _write_out(c, sid, acc, cnt_a, cnt_b)


def _dot_t(x, w):
    return lax.dot_general(x, w, (((1,), (1,)), ((), ())),
                           preferred_element_type=jnp.float32)


def _tc1_body(x_ref, w0_ref, b0_ref, w1_ref, b1_ref, lo_ref, hi_ref):
    h = jnp.maximum(_dot_t(x_ref[...], w0_ref[...]) + b0_ref[...], 0.0)
    xt = _dot_t(h, w1_ref[...]) + b1_ref[...]
    lo_ref[...] = xt[:, :DH]
    hi_ref[...] = xt[:, DH:]


def _tc1(x, in_W, in_b, l0_W, l0_b):
    return pl.pallas_call(
        _tc1_body,
        grid=(GRID,),
        in_specs=[
            pl.BlockSpec((BN, D), lambda i: (i, 0)),
            pl.BlockSpec((D, D), lambda i: (0, 0)),
            pl.BlockSpec((1, D), lambda i: (0, 0)),
            pl.BlockSpec((D, D), lambda i: (0, 0)),
            pl.BlockSpec((1, D), lambda i: (0, 0)),
        ],
        out_specs=[pl.BlockSpec((BN, DH), lambda i: (i, 0))] * 2,
        out_shape=[jax.ShapeDtypeStruct((N, DH), jnp.float32)] * 2,
    )(x, in_W, in_b, l0_W, l0_b)


def _mean_sub(a_ref, ca_ref, cb_ref, xlo_ref, xhi_ref, slo_ref, shi_ref,
              ntW_ref, ntb_ref):
    sig = jax.nn.sigmoid(a_ref[0, 0])
    cnt = ca_ref[...][:, :1] + cb_ref[...][:, :1]
    scale = 1.0 / jnp.maximum(cnt, 1.0)
    ind = (cnt > 0.0).astype(jnp.float32)
    g = jnp.concatenate([slo_ref[...], shi_ref[...]], axis=1) * scale
    mean = _dot_t(g, ntW_ref[...]) + ind * ntb_ref[...]
    xt = jnp.concatenate([xlo_ref[...], xhi_ref[...]], axis=1)
    return jnp.maximum(xt - sig * mean, 0.0)


def _tc2_body(a_ref, ca_ref, cb_ref, xlo_ref, xhi_ref, slo_ref, shi_ref,
              ntW_ref, ntb_ref, feW_ref, feb_ref, l1W_ref, l1b_ref,
              olo_ref, ohi_ref):
    h1 = _mean_sub(a_ref, ca_ref, cb_ref, xlo_ref, xhi_ref, slo_ref, shi_ref,
                   ntW_ref, ntb_ref)
    h2 = jnp.maximum(_dot_t(h1, feW_ref[...]) + feb_ref[...], 0.0)
    xt1 = _dot_t(h2, l1W_ref[...]) + l1b_ref[...]
    olo_ref[...] = xt1[:, :DH]
    ohi_ref[...] = xt1[:, DH:]


_SPEC_SCALAR = pl.BlockSpec(memory_space=pltpu.SMEM)
_SPEC_CNT = pl.BlockSpec((BN, CNT_W), lambda i: (i, 0))
_SPEC_HALF = pl.BlockSpec((BN, DH), lambda i: (i, 0))
_SPEC_W = pl.BlockSpec((D, D), lambda i: (0, 0))
_SPEC_B = pl.BlockSpec((1, D), lambda i: (0, 0))


def _tc2(a, cnt_a, cnt_b, xt_lo, xt_hi, s_lo, s_hi,
         nt_W, nt_b, fe_W, fe_b, l1_W, l1_b):
    return pl.pallas_call(
        _tc2_body,
        grid=(GRID,),
        in_specs=[_SPEC_SCALAR, _SPEC_CNT, _SPEC_CNT,
                  _SPEC_HALF, _SPEC_HALF, _SPEC_HALF, _SPEC_HALF,
                  _SPEC_W, _SPEC_B, _SPEC_W, _SPEC_B, _SPEC_W, _SPEC_B],
        out_specs=[_SPEC_HALF] * 2,
        out_shape=[jax.ShapeDtypeStruct((N, DH), jnp.float32)] * 2,
    )(a, cnt_a, cnt_b, xt_lo, xt_hi, s_lo, s_hi,
      nt_W, nt_b, fe_W, fe_b, l1_W, l1_b)


def _tc3_body(a_ref, ca_ref, cb_ref, xlo_ref, xhi_ref, slo_ref, shi_ref,
              ntW_ref, ntb_ref, oW_ref, ob_ref, o_ref):
    h3 = _mean_sub(a_ref, ca_ref, cb_ref, xlo_ref, xhi_ref, slo_ref, shi_ref,
                   ntW_ref, ntb_ref)
    o_ref[...] = _dot_t(h3, oW_ref[...]) + ob_ref[...]


def _tc3(a, cnt_a, cnt_b, xt_lo, xt_hi, s_lo, s_hi, nt_W, nt_b, out_W, out_b):
    return pl.pallas_call(
        _tc3_body,
        grid=(GRID,),
        in_specs=[_SPEC_SCALAR, _SPEC_CNT, _SPEC_CNT,
                  _SPEC_HALF, _SPEC_HALF, _SPEC_HALF, _SPEC_HALF,
                  _SPEC_W, _SPEC_B, _SPEC_W, _SPEC_B],
        out_specs=pl.BlockSpec((BN, D), lambda i: (i, 0)),
        out_shape=jax.ShapeDtypeStruct((N, D), jnp.float32),
    )(a, cnt_a, cnt_b, xt_lo, xt_hi, s_lo, s_hi, nt_W, nt_b, out_W, out_b)


def kernel(x, edge_index, in_W, in_b, l0_W, l0_b, nt0_W, nt0_b, a0,
           l1_W, l1_b, nt1_W, nt1_b, a1, fe0_W, fe0_b, out_W, out_b):
    row = edge_index[0]
    col = edge_index[1]
    pad = E_PAD - E
    row_p = jnp.concatenate([row, jnp.zeros((pad,), jnp.int32)])
    col_p = jnp.concatenate([col, jnp.full((pad,), N, jnp.int32)])

    def b2(v):
        return v.reshape(1, D)

    a0r = jnp.reshape(a0, (1, 1))
    a1r = jnp.reshape(a1, (1, 1))

    xt_lo, xt_hi = _tc1(x, in_W, b2(in_b), l0_W, b2(l0_b))
    cnt_a, cnt_b = _cnt(col_p)
    s0_lo, s0_hi = _segsum(xt_lo, xt_hi, row_p, col_p)
    xt1_lo, xt1_hi = _tc2(a0r, cnt_a, cnt_b, xt_lo, xt_hi, s0_lo, s0_hi,
                          nt0_W, b2(nt0_b), fe0_W, b2(fe0_b), l1_W, b2(l1_b))
    s1_lo, s1_hi = _segsum(xt1_lo, xt1_hi, row_p, col_p)
    return _tc3(a1r, cnt_a, cnt_b, xt1_lo, xt1_hi, s1_lo, s1_hi,
                nt1_W, b2(nt1_b), out_W, b2(out_b))


# speedup vs baseline: 3.0702x; 1.2124x over previous
import functools

import jax
import jax.numpy as jnp
from jax import lax
from jax.experimental import pallas as pl
from jax.experimental.pallas import tpu as pltpu
from jax.experimental.pallas import tpu_sc as plsc

N = 10000
E = 160000
D = 256
DH = 128
E_PAD = 163840
N_ACC = 10112
ZSPAN = 632
OSPAN = 632
OSPAN_LAST = N - 15 * OSPAN
CHUNK = 128
SEG_CHUNKS = E_PAD // 16 // CHUNK
CNT_CHUNKS = E_PAD // 32 // CHUNK
CNT_W = 128

BN = 400
GRID = N // BN

_mesh = plsc.VectorSubcoreMesh(core_axis_name="c", subcore_axis_name="s")


NBUF = 2
NGRP = SEG_CHUNKS // NBUF


@functools.partial(
    pl.kernel,
    mesh=_mesh,
    out_type=[
        jax.ShapeDtypeStruct((N, DH), jnp.float32),
        jax.ShapeDtypeStruct((N, DH), jnp.float32),
    ],
    scratch_types=[
        pltpu.VMEM_SHARED((N_ACC, DH), jnp.float32),
    ] + [pltpu.VMEM((CHUNK, DH), jnp.float32) for _ in range(NBUF)]
      + [pltpu.VMEM((CHUNK,), jnp.int32) for _ in range(2 * NBUF)] + [
        pltpu.SemaphoreType.DMA for _ in range(3 * NBUF)
    ],
)
def _segsum(xt_lo, xt_hi, row_hbm, col_hbm, s_lo, s_hi,
            acc, g0, g1, r0, r1, c0, c1, gs0, gs1, ss0, ss1, is0, is1):
    c = lax.axis_index("c")
    sid = lax.axis_index("s")
    gbufs = (g0, g1)
    rbufs = (r0, r1)
    cbufs = (c0, c1)
    gsems = (gs0, gs1)
    ssems = (ss0, ss1)
    isems = (is0, is1)

    def _zrow(i, carry):
        for j in range(DH // 16):
            g0[i, pl.ds(j * 16, 16)] = jnp.zeros((16,), jnp.float32)
        return carry
    lax.fori_loop(0, CHUNK, _zrow, 0)
    zbase = sid * ZSPAN
    for k in range((ZSPAN + CHUNK - 1) // CHUNK):
        n = min(CHUNK, ZSPAN - k * CHUNK)
        pltpu.sync_copy(g0.at[pl.ds(0, n)],
                        acc.at[pl.ds(zbase + k * CHUNK, n)])
    plsc.subcore_barrier()

    ebase = sid * (E_PAD // 16)

    def _pf_idx(b, ch):
        off = ebase + ch * CHUNK
        pltpu.async_copy(row_hbm.at[pl.ds(off, CHUNK)], rbufs[b], isems[b])
        pltpu.async_copy(col_hbm.at[pl.ds(off, CHUNK)], cbufs[b], isems[b])

    def _wait_idx(b):
        pltpu.make_async_copy(row_hbm.at[pl.ds(0, CHUNK)], rbufs[b],
                              isems[b]).wait()
        pltpu.make_async_copy(col_hbm.at[pl.ds(0, CHUNK)], cbufs[b],
                              isems[b]).wait()

    def _edge_loop(table):
        def _gather(b):
            pltpu.async_copy(table.at[rbufs[b]], gbufs[b], gsems[b])

        def _wait_gather(b):
            pltpu.make_async_copy(table.at[pl.ds(0, CHUNK)], gbufs[b],
                                  gsems[b]).wait()

        def _scatter(b):
            pltpu.async_copy(gbufs[b], acc.at[cbufs[b]], ssems[b], add=True)

        def _wait_scatter(b):
            pltpu.make_async_copy(table.at[pl.ds(0, CHUNK)], gbufs[b],
                                  ssems[b]).wait()

        for b in range(NBUF):
            _pf_idx(b, b)
        for b in range(NBUF):
            _wait_idx(b)
            _gather(b)

        def _body(i, carry):
            g = i * NBUF
            for b in range(NBUF):
                _wait_gather(b)
                _scatter(b)
            for b in range(NBUF):
                _wait_scatter(b)
                _pf_idx(b, g + NBUF + b)
                _wait_idx(b)
                _gather(b)
            return carry
        lax.fori_loop(0, NGRP - 1, _body, 0)

        for b in range(NBUF):
            _wait_gather(b)
            _scatter(b)
        for b in range(NBUF):
            _wait_scatter(b)

    @pl.when(c == 0)
    def _():
        _edge_loop(xt_lo)

    @pl.when(c == 1)
    def _():
        _edge_loop(xt_hi)

    plsc.subcore_barrier()
    _write_out(c, sid, acc, s_lo, s_hi)


def _write_out(c, sid, acc, out0, out1):
    obase = sid * OSPAN

    @pl.when((c == 0) & (sid < 15))
    def _():
        pltpu.sync_copy(acc.at[pl.ds(obase, OSPAN)],
                        out0.at[pl.ds(obase, OSPAN)])

    @pl.when((c == 0) & (sid == 15))
    def _():
        pltpu.sync_copy(acc.at[pl.ds(15 * OSPAN, OSPAN_LAST)],
                        out0.at[pl.ds(15 * OSPAN, OSPAN_LAST)])

    @pl.when((c == 1) & (sid < 15))
    def _():
        pltpu.sync_copy(acc.at[pl.ds(obase, OSPAN)],
                        out1.at[pl.ds(obase, OSPAN)])

    @pl.when((c == 1) & (sid == 15))
    def _():
        pltpu.sync_copy(acc.at[pl.ds(15 * OSPAN, OSPAN_LAST)],
                        out1.at[pl.ds(15 * OSPAN, OSPAN_LAST)])


@functools.partial(
    pl.kernel,
    mesh=_mesh,
    out_type=[
        jax.ShapeDtypeStruct((N, CNT_W), jnp.float32),
        jax.ShapeDtypeStruct((N, CNT_W), jnp.float32),
    ],
    scratch_types=[
        pltpu.VMEM_SHARED((N_ACC, CNT_W), jnp.float32),
        pltpu.VMEM((CNT_CHUNKS, CHUNK), jnp.int32),
        pltpu.VMEM((CHUNK, CNT_W), jnp.float32),
        pltpu.VMEM((CHUNK, CNT_W), jnp.float32),
    ],
)
def _cnt(col_hbm, cnt_a, cnt_b, acc, col_idx, ones_buf, zbuf):
    c = lax.axis_index("c")
    sid = lax.axis_index("s")

    wid = sid * 2 + c
    cbase = pl.multiple_of(wid * CNT_CHUNKS, 8)
    pltpu.sync_copy(col_hbm.at[pl.ds(cbase, CNT_CHUNKS)], col_idx)

    def _fill(i, carry):
        for j in range(CNT_W // 16):
            ones_buf[i, pl.ds(j * 16, 16)] = jnp.ones((16,), jnp.float32)
            zbuf[i, pl.ds(j * 16, 16)] = jnp.zeros((16,), jnp.float32)
        return carry
    lax.fori_loop(0, CHUNK, _fill, 0)
    zbase = sid * ZSPAN
    for k in range((ZSPAN + CHUNK - 1) // CHUNK):
        n = min(CHUNK, ZSPAN - k * CHUNK)
        pltpu.sync_copy(zbuf.at[pl.ds(0, n)],
                        acc.at[pl.ds(zbase + k * CHUNK, n)])
    plsc.subcore_barrier()

    def _body(ch, carry):
        pltpu.sync_copy(ones_buf, acc.at[col_idx.at[ch]], add=True)
        return carry
    lax.fori_loop(0, CNT_CHUNKS, _body, 0)

    plsc.subcore_barrier()
    _write_out(c, sid, acc, cnt_a, cnt_b)


def _dot_t(x, w):
    return lax.dot_general(x, w, (((1,), (1,)), ((), ())),
                           preferred_element_type=jnp.float32)


def _tc1_body(x_ref, w0_ref, b0_ref, w1_ref, b1_ref, lo_ref, hi_ref):
    h = jnp.maximum(_dot_t(x_ref[...], w0_ref[...]) + b0_ref[...], 0.0)
    xt = _dot_t(h, w1_ref[...]) + b1_ref[...]
    lo_ref[...] = xt[:, :DH]
    hi_ref[...] = xt[:, DH:]


def _tc1(x, in_W, in_b, l0_W, l0_b):
    return pl.pallas_call(
        _tc1_body,
        grid=(GRID,),
        in_specs=[
            pl.BlockSpec((BN, D), lambda i: (i, 0)),
            pl.BlockSpec((D, D), lambda i: (0, 0)),
            pl.BlockSpec((1, D), lambda i: (0, 0)),
            pl.BlockSpec((D, D), lambda i: (0, 0)),
            pl.BlockSpec((1, D), lambda i: (0, 0)),
        ],
        out_specs=[pl.BlockSpec((BN, DH), lambda i: (i, 0))] * 2,
        out_shape=[jax.ShapeDtypeStruct((N, DH), jnp.float32)] * 2,
    )(x, in_W, in_b, l0_W, l0_b)


def _mean_sub(a_ref, ca_ref, cb_ref, xlo_ref, xhi_ref, slo_ref, shi_ref,
              ntW_ref, ntb_ref):
    sig = jax.nn.sigmoid(a_ref[0, 0])
    cnt = ca_ref[...][:, :1] + cb_ref[...][:, :1]
    scale = 1.0 / jnp.maximum(cnt, 1.0)
    ind = (cnt > 0.0).astype(jnp.float32)
    g = jnp.concatenate([slo_ref[...], shi_ref[...]], axis=1) * scale
    mean = _dot_t(g, ntW_ref[...]) + ind * ntb_ref[...]
    xt = jnp.concatenate([xlo_ref[...], xhi_ref[...]], axis=1)
    return jnp.maximum(xt - sig * mean, 0.0)


def _tc2_body(a_ref, ca_ref, cb_ref, xlo_ref, xhi_ref, slo_ref, shi_ref,
              ntW_ref, ntb_ref, feW_ref, feb_ref, l1W_ref, l1b_ref,
              olo_ref, ohi_ref):
    h1 = _mean_sub(a_ref, ca_ref, cb_ref, xlo_ref, xhi_ref, slo_ref, shi_ref,
                   ntW_ref, ntb_ref)
    h2 = jnp.maximum(_dot_t(h1, feW_ref[...]) + feb_ref[...], 0.0)
    xt1 = _dot_t(h2, l1W_ref[...]) + l1b_ref[...]
    olo_ref[...] = xt1[:, :DH]
    ohi_ref[...] = xt1[:, DH:]


_SPEC_SCALAR = pl.BlockSpec(memory_space=pltpu.SMEM)
_SPEC_CNT = pl.BlockSpec((BN, CNT_W), lambda i: (i, 0))
_SPEC_HALF = pl.BlockSpec((BN, DH), lambda i: (i, 0))
_SPEC_W = pl.BlockSpec((D, D), lambda i: (0, 0))
_SPEC_B = pl.BlockSpec((1, D), lambda i: (0, 0))


def _tc2(a, cnt_a, cnt_b, xt_lo, xt_hi, s_lo, s_hi,
         nt_W, nt_b, fe_W, fe_b, l1_W, l1_b):
    return pl.pallas_call(
        _tc2_body,
        grid=(GRID,),
        in_specs=[_SPEC_SCALAR, _SPEC_CNT, _SPEC_CNT,
                  _SPEC_HALF, _SPEC_HALF, _SPEC_HALF, _SPEC_HALF,
                  _SPEC_W, _SPEC_B, _SPEC_W, _SPEC_B, _SPEC_W, _SPEC_B],
        out_specs=[_SPEC_HALF] * 2,
        out_shape=[jax.ShapeDtypeStruct((N, DH), jnp.float32)] * 2,
    )(a, cnt_a, cnt_b, xt_lo, xt_hi, s_lo, s_hi,
      nt_W, nt_b, fe_W, fe_b, l1_W, l1_b)


def _tc3_body(a_ref, ca_ref, cb_ref, xlo_ref, xhi_ref, slo_ref, shi_ref,
              ntW_ref, ntb_ref, oW_ref, ob_ref, o_ref):
    h3 = _mean_sub(a_ref, ca_ref, cb_ref, xlo_ref, xhi_ref, slo_ref, shi_ref,
                   ntW_ref, ntb_ref)
    o_ref[...] = _dot_t(h3, oW_ref[...]) + ob_ref[...]


def _tc3(a, cnt_a, cnt_b, xt_lo, xt_hi, s_lo, s_hi, nt_W, nt_b, out_W, out_b):
    return pl.pallas_call(
        _tc3_body,
        grid=(GRID,),
        in_specs=[_SPEC_SCALAR, _SPEC_CNT, _SPEC_CNT,
                  _SPEC_HALF, _SPEC_HALF, _SPEC_HALF, _SPEC_HALF,
                  _SPEC_W, _SPEC_B, _SPEC_W, _SPEC_B],
        out_specs=pl.BlockSpec((BN, D), lambda i: (i, 0)),
        out_shape=jax.ShapeDtypeStruct((N, D), jnp.float32),
    )(a, cnt_a, cnt_b, xt_lo, xt_hi, s_lo, s_hi, nt_W, nt_b, out_W, out_b)


def kernel(x, edge_index, in_W, in_b, l0_W, l0_b, nt0_W, nt0_b, a0,
           l1_W, l1_b, nt1_W, nt1_b, a1, fe0_W, fe0_b, out_W, out_b):
    row = edge_index[0]
    col = edge_index[1]
    pad = E_PAD - E
    row_p = jnp.concatenate([row, jnp.zeros((pad,), jnp.int32)])
    col_p = jnp.concatenate([col, jnp.full((pad,), N, jnp.int32)])
    col_2d = col_p.reshape(E_PAD // CHUNK, CHUNK)

    def b2(v):
        return v.reshape(1, D)

    a0r = jnp.reshape(a0, (1, 1))
    a1r = jnp.reshape(a1, (1, 1))

    xt_lo, xt_hi = _tc1(x, in_W, b2(in_b), l0_W, b2(l0_b))
    cnt_a, cnt_b = _cnt(col_2d)
    s0_lo, s0_hi = _segsum(xt_lo, xt_hi, row_p, col_p)
    xt1_lo, xt1_hi = _tc2(a0r, cnt_a, cnt_b, xt_lo, xt_hi, s0_lo, s0_hi,
                          nt0_W, b2(nt0_b), fe0_W, b2(fe0_b), l1_W, b2(l1_b))
    s1_lo, s1_hi = _segsum(xt1_lo, xt1_hi, row_p, col_p)
    return _tc3(a1r, cnt_a, cnt_b, xt1_lo, xt1_hi, s1_lo, s1_hi,
                nt1_W, b2(nt1_b), out_W, b2(out_b))


# speedup vs baseline: 3.2542x; 1.0600x over previous
import functools

import jax
import jax.numpy as jnp
from jax import lax
from jax.experimental import pallas as pl
from jax.experimental.pallas import tpu as pltpu
from jax.experimental.pallas import tpu_sc as plsc

N = 10000
E = 160000
D = 256
DH = 128
E_PAD = 163840
N_ACC = 10112
ZSPAN = 632
OSPAN = 632
OSPAN_LAST = N - 15 * OSPAN
CHUNK = 128
SEG_CHUNKS = E_PAD // 16 // CHUNK
CNT_CHUNKS = E_PAD // 32 // CHUNK
CNT_W = 128

BN = 400
GRID = N // BN

_mesh = plsc.VectorSubcoreMesh(core_axis_name="c", subcore_axis_name="s")


NBUF = 2
NGRP = SEG_CHUNKS // NBUF


@functools.partial(
    pl.kernel,
    mesh=_mesh,
    out_type=[
        jax.ShapeDtypeStruct((N, DH), jnp.float32),
        jax.ShapeDtypeStruct((N, DH), jnp.float32),
    ],
    scratch_types=[
        pltpu.VMEM_SHARED((N_ACC, DH), jnp.float32),
        pltpu.VMEM((SEG_CHUNKS, CHUNK), jnp.int32),
    ] + [pltpu.VMEM((CHUNK, DH), jnp.float32) for _ in range(NBUF)]
      + [pltpu.VMEM((CHUNK,), jnp.int32) for _ in range(NBUF)] + [
        pltpu.SemaphoreType.DMA for _ in range(3 * NBUF)
    ],
)
def _segsum(xt_lo, xt_hi, row_hbm, col_hbm, s_lo, s_hi,
            acc, col_idx, g0, g1, r0, r1, gs0, gs1, ss0, ss1, is0, is1):
    c = lax.axis_index("c")
    sid = lax.axis_index("s")
    gbufs = (g0, g1)
    rbufs = (r0, r1)
    gsems = (gs0, gs1)
    ssems = (ss0, ss1)
    isems = (is0, is1)

    pltpu.sync_copy(col_hbm.at[pl.ds(sid * SEG_CHUNKS, SEG_CHUNKS)], col_idx)

    def _zrow(i, carry):
        for j in range(DH // 16):
            g0[i, pl.ds(j * 16, 16)] = jnp.zeros((16,), jnp.float32)
        return carry
    lax.fori_loop(0, CHUNK, _zrow, 0)
    zbase = sid * ZSPAN
    for k in range((ZSPAN + CHUNK - 1) // CHUNK):
        n = min(CHUNK, ZSPAN - k * CHUNK)
        pltpu.sync_copy(g0.at[pl.ds(0, n)],
                        acc.at[pl.ds(zbase + k * CHUNK, n)])
    plsc.subcore_barrier()

    ebase = sid * (E_PAD // 16)

    def _pf_row(b, ch):
        pltpu.async_copy(row_hbm.at[pl.ds(ebase + ch * CHUNK, CHUNK)],
                         rbufs[b], isems[b])

    def _wait_row(b):
        pltpu.make_async_copy(row_hbm.at[pl.ds(0, CHUNK)], rbufs[b],
                              isems[b]).wait()

    def _edge_loop(table):
        def _gather(b):
            pltpu.async_copy(table.at[rbufs[b]], gbufs[b], gsems[b])

        def _wait_gather(b):
            pltpu.make_async_copy(table.at[pl.ds(0, CHUNK)], gbufs[b],
                                  gsems[b]).wait()

        def _scatter(b, ch):
            pltpu.async_copy(gbufs[b], acc.at[col_idx.at[ch]], ssems[b],
                             add=True)

        def _wait_scatter(b):
            pltpu.make_async_copy(table.at[pl.ds(0, CHUNK)], gbufs[b],
                                  ssems[b]).wait()

        for b in range(NBUF):
            _pf_row(b, b)
        for b in range(NBUF):
            _wait_row(b)
            _gather(b)

        def _body(i, carry):
            g = i * NBUF
            for b in range(NBUF):
                _wait_gather(b)
                _pf_row(b, g + NBUF + b)
                _scatter(b, g + b)
            for b in range(NBUF):
                _wait_scatter(b)
                _wait_row(b)
                _gather(b)
            return carry
        lax.fori_loop(0, NGRP - 1, _body, 0)

        g = (NGRP - 1) * NBUF
        for b in range(NBUF):
            _wait_gather(b)
            _scatter(b, g + b)
        for b in range(NBUF):
            _wait_scatter(b)

    @pl.when(c == 0)
    def _():
        _edge_loop(xt_lo)

    @pl.when(c == 1)
    def _():
        _edge_loop(xt_hi)

    plsc.subcore_barrier()
    _write_out(c, sid, acc, s_lo, s_hi)


def _write_out(c, sid, acc, out0, out1):
    obase = sid * OSPAN

    @pl.when((c == 0) & (sid < 15))
    def _():
        pltpu.sync_copy(acc.at[pl.ds(obase, OSPAN)],
                        out0.at[pl.ds(obase, OSPAN)])

    @pl.when((c == 0) & (sid == 15))
    def _():
        pltpu.sync_copy(acc.at[pl.ds(15 * OSPAN, OSPAN_LAST)],
                        out0.at[pl.ds(15 * OSPAN, OSPAN_LAST)])

    @pl.when((c == 1) & (sid < 15))
    def _():
        pltpu.sync_copy(acc.at[pl.ds(obase, OSPAN)],
                        out1.at[pl.ds(obase, OSPAN)])

    @pl.when((c == 1) & (sid == 15))
    def _():
        pltpu.sync_copy(acc.at[pl.ds(15 * OSPAN, OSPAN_LAST)],
                        out1.at[pl.ds(15 * OSPAN, OSPAN_LAST)])


@functools.partial(
    pl.kernel,
    mesh=_mesh,
    out_type=[
        jax.ShapeDtypeStruct((N, CNT_W), jnp.float32),
        jax.ShapeDtypeStruct((N, CNT_W), jnp.float32),
    ],
    scratch_types=[
        pltpu.VMEM_SHARED((N_ACC, CNT_W), jnp.float32),
        pltpu.VMEM((CNT_CHUNKS, CHUNK), jnp.int32),
        pltpu.VMEM((CHUNK, CNT_W), jnp.float32),
        pltpu.VMEM((CHUNK, CNT_W), jnp.float32),
    ],
)
def _cnt(col_hbm, cnt_a, cnt_b, acc, col_idx, ones_buf, zbuf):
    c = lax.axis_index("c")
    sid = lax.axis_index("s")

    wid = sid * 2 + c
    cbase = pl.multiple_of(wid * CNT_CHUNKS, 8)
    pltpu.sync_copy(col_hbm.at[pl.ds(cbase, CNT_CHUNKS)], col_idx)

    def _fill(i, carry):
        for j in range(CNT_W // 16):
            ones_buf[i, pl.ds(j * 16, 16)] = jnp.ones((16,), jnp.float32)
            zbuf[i, pl.ds(j * 16, 16)] = jnp.zeros((16,), jnp.float32)
        return carry
    lax.fori_loop(0, CHUNK, _fill, 0)
    zbase = sid * ZSPAN
    for k in range((ZSPAN + CHUNK - 1) // CHUNK):
        n = min(CHUNK, ZSPAN - k * CHUNK)
        pltpu.sync_copy(zbuf.at[pl.ds(0, n)],
                        acc.at[pl.ds(zbase + k * CHUNK, n)])
    plsc.subcore_barrier()

    def _body(ch, carry):
        pltpu.sync_copy(ones_buf, acc.at[col_idx.at[ch]], add=True)
        return carry
    lax.fori_loop(0, CNT_CHUNKS, _body, 0)

    plsc.subcore_barrier()
    _write_out(c, sid, acc, cnt_a, cnt_b)


def _dot_t(x, w):
    return lax.dot_general(x, w, (((1,), (1,)), ((), ())),
                           preferred_element_type=jnp.float32)


def _tc1_body(x_ref, w0_ref, b0_ref, w1_ref, b1_ref, lo_ref, hi_ref):
    h = jnp.maximum(_dot_t(x_ref[...], w0_ref[...]) + b0_ref[...], 0.0)
    xt = _dot_t(h, w1_ref[...]) + b1_ref[...]
    lo_ref[...] = xt[:, :DH]
    hi_ref[...] = xt[:, DH:]


def _tc1(x, in_W, in_b, l0_W, l0_b):
    return pl.pallas_call(
        _tc1_body,
        grid=(GRID,),
        in_specs=[
            pl.BlockSpec((BN, D), lambda i: (i, 0)),
            pl.BlockSpec((D, D), lambda i: (0, 0)),
            pl.BlockSpec((1, D), lambda i: (0, 0)),
            pl.BlockSpec((D, D), lambda i: (0, 0)),
            pl.BlockSpec((1, D), lambda i: (0, 0)),
        ],
        out_specs=[pl.BlockSpec((BN, DH), lambda i: (i, 0))] * 2,
        out_shape=[jax.ShapeDtypeStruct((N, DH), jnp.float32)] * 2,
    )(x, in_W, in_b, l0_W, l0_b)


def _mean_sub(a_ref, ca_ref, cb_ref, xlo_ref, xhi_ref, slo_ref, shi_ref,
              ntW_ref, ntb_ref):
    sig = jax.nn.sigmoid(a_ref[0, 0])
    cnt = ca_ref[...][:, :1] + cb_ref[...][:, :1]
    scale = 1.0 / jnp.maximum(cnt, 1.0)
    ind = (cnt > 0.0).astype(jnp.float32)
    g = jnp.concatenate([slo_ref[...], shi_ref[...]], axis=1) * scale
    mean = _dot_t(g, ntW_ref[...]) + ind * ntb_ref[...]
    xt = jnp.concatenate([xlo_ref[...], xhi_ref[...]], axis=1)
    return jnp.maximum(xt - sig * mean, 0.0)


def _tc2_body(a_ref, ca_ref, cb_ref, xlo_ref, xhi_ref, slo_ref, shi_ref,
              ntW_ref, ntb_ref, feW_ref, feb_ref, l1W_ref, l1b_ref,
              olo_ref, ohi_ref):
    h1 = _mean_sub(a_ref, ca_ref, cb_ref, xlo_ref, xhi_ref, slo_ref, shi_ref,
                   ntW_ref, ntb_ref)
    h2 = jnp.maximum(_dot_t(h1, feW_ref[...]) + feb_ref[...], 0.0)
    xt1 = _dot_t(h2, l1W_ref[...]) + l1b_ref[...]
    olo_ref[...] = xt1[:, :DH]
    ohi_ref[...] = xt1[:, DH:]


_SPEC_SCALAR = pl.BlockSpec(memory_space=pltpu.SMEM)
_SPEC_CNT = pl.BlockSpec((BN, CNT_W), lambda i: (i, 0))
_SPEC_HALF = pl.BlockSpec((BN, DH), lambda i: (i, 0))
_SPEC_W = pl.BlockSpec((D, D), lambda i: (0, 0))
_SPEC_B = pl.BlockSpec((1, D), lambda i: (0, 0))


def _tc2(a, cnt_a, cnt_b, xt_lo, xt_hi, s_lo, s_hi,
         nt_W, nt_b, fe_W, fe_b, l1_W, l1_b):
    return pl.pallas_call(
        _tc2_body,
        grid=(GRID,),
        in_specs=[_SPEC_SCALAR, _SPEC_CNT, _SPEC_CNT,
                  _SPEC_HALF, _SPEC_HALF, _SPEC_HALF, _SPEC_HALF,
                  _SPEC_W, _SPEC_B, _SPEC_W, _SPEC_B, _SPEC_W, _SPEC_B],
        out_specs=[_SPEC_HALF] * 2,
        out_shape=[jax.ShapeDtypeStruct((N, DH), jnp.float32)] * 2,
    )(a, cnt_a, cnt_b, xt_lo, xt_hi, s_lo, s_hi,
      nt_W, nt_b, fe_W, fe_b, l1_W, l1_b)


def _tc3_body(a_ref, ca_ref, cb_ref, xlo_ref, xhi_ref, slo_ref, shi_ref,
              ntW_ref, ntb_ref, oW_ref, ob_ref, o_ref):
    h3 = _mean_sub(a_ref, ca_ref, cb_ref, xlo_ref, xhi_ref, slo_ref, shi_ref,
                   ntW_ref, ntb_ref)
    o_ref[...] = _dot_t(h3, oW_ref[...]) + ob_ref[...]


def _tc3(a, cnt_a, cnt_b, xt_lo, xt_hi, s_lo, s_hi, nt_W, nt_b, out_W, out_b):
    return pl.pallas_call(
        _tc3_body,
        grid=(GRID,),
        in_specs=[_SPEC_SCALAR, _SPEC_CNT, _SPEC_CNT,
                  _SPEC_HALF, _SPEC_HALF, _SPEC_HALF, _SPEC_HALF,
                  _SPEC_W, _SPEC_B, _SPEC_W, _SPEC_B],
        out_specs=pl.BlockSpec((BN, D), lambda i: (i, 0)),
        out_shape=jax.ShapeDtypeStruct((N, D), jnp.float32),
    )(a, cnt_a, cnt_b, xt_lo, xt_hi, s_lo, s_hi, nt_W, nt_b, out_W, out_b)


def kernel(x, edge_index, in_W, in_b, l0_W, l0_b, nt0_W, nt0_b, a0,
           l1_W, l1_b, nt1_W, nt1_b, a1, fe0_W, fe0_b, out_W, out_b):
    row = edge_index[0]
    col = edge_index[1]
    pad = E_PAD - E
    row_p = jnp.concatenate([row, jnp.zeros((pad,), jnp.int32)])
    col_p = jnp.concatenate([col, jnp.full((pad,), N, jnp.int32)])
    col_2d = col_p.reshape(E_PAD // CHUNK, CHUNK)

    def b2(v):
        return v.reshape(1, D)

    a0r = jnp.reshape(a0, (1, 1))
    a1r = jnp.reshape(a1, (1, 1))

    xt_lo, xt_hi = _tc1(x, in_W, b2(in_b), l0_W, b2(l0_b))
    cnt_a, cnt_b = _cnt(col_2d)
    s0_lo, s0_hi = _segsum(xt_lo, xt_hi, row_p, col_2d)
    xt1_lo, xt1_hi = _tc2(a0r, cnt_a, cnt_b, xt_lo, xt_hi, s0_lo, s0_hi,
                          nt0_W, b2(nt0_b), fe0_W, b2(fe0_b), l1_W, b2(l1_b))
    s1_lo, s1_hi = _segsum(xt1_lo, xt1_hi, row_p, col_2d)
    return _tc3(a1r, cnt_a, cnt_b, xt1_lo, xt1_hi, s1_lo, s1_hi,
                nt1_W, b2(nt1_b), out_W, b2(out_b))
